# baseline (speedup 1.0000x reference)
import jax
import jax.numpy as jnp
from jax import lax
from jax.experimental import pallas as pl
from jax.experimental.pallas import tpu as pltpu
from jax.experimental.pallas import tpu_sc as plsc

B = 16384
D = 32
NC = 2
NS = 16
NW = NC * NS
CHUNK = B // NW
QROWS = CHUNK // 128
NITEMS = 1_000_000


def _body(user_hbm, item_hbm, tbl_hbm, sig_hbm, ub_hbm, ib_hbm,
          out_hbm, uidx, iidx, iidx2, urows, vrows, ud, vtd, ubv, ibv, sig,
          outv, sem):
  wid = lax.axis_index("s") * NC + lax.axis_index("c")
  r0 = wid * QROWS
  base = wid * CHUNK
  iota = lax.iota(jnp.int32, 16)

  pltpu.sync_copy(user_hbm.at[pl.ds(r0, QROWS)], uidx)
  pltpu.sync_copy(item_hbm.at[pl.ds(r0, QROWS)], iidx)
  pltpu.sync_copy(sig_hbm, sig)

  def shift(q, _):
    for c in range(0, 128, 16):
      iidx2[q, pl.ds(c, 16)] = iidx[q, pl.ds(c, 16)] + NITEMS
    return 0

  lax.fori_loop(0, QROWS, shift, 0)

  copies = []
  for q in range(QROWS):
    copies.append(pltpu.async_copy(ub_hbm.at[uidx.at[q]], ubv.at[q], sem))
    copies.append(pltpu.async_copy(ib_hbm.at[iidx.at[q]], ibv.at[q], sem))
    copies.append(
        pltpu.async_copy(tbl_hbm.at[uidx.at[q]],
                         urows.at[pl.ds(q * 128, 128)], sem))
    copies.append(
        pltpu.async_copy(tbl_hbm.at[iidx2.at[q]],
                         vrows.at[pl.ds(q * 128, 128)], sem))
  for cp in copies:
    cp.wait()

  dvec = iota * CHUNK

  def transpose(j, _):
    for h in range(2):
      idx = dvec + (h * 16 * CHUNK + j)
      plsc.store_scatter(ud, [idx], urows[j, pl.ds(h * 16, 16)])
      plsc.store_scatter(vtd, [idx], vrows[j, pl.ds(h * 16, 16)])
    return 0

  lax.fori_loop(0, CHUNK, transpose, 0)

  def compute(jc, _):
    row = jc // 8
    col = (jc % 8) * 16
    sig_lo = sig[pl.ds(0, 16)]
    sig_hi = sig[pl.ds(16, 16)]
    acc = ubv[row, pl.ds(col, 16)] + ibv[row, pl.ds(col, 16)]
    for d in range(D):
      sig_d = sig_lo[d] if d < 16 else sig_hi[d - 16]
      vt_chunk = vtd[pl.ds(d * CHUNK + jc * 16, 16)]
      u_chunk = ud[pl.ds(d * CHUNK + jc * 16, 16)]
      acc = acc + (sig_d * vt_chunk) * u_chunk
    outv[pl.ds(jc * 16, 16)] = acc
    return 0

  lax.fori_loop(0, CHUNK // 16, compute, 0)

  pltpu.sync_copy(outv, out_hbm.at[pl.ds(base, CHUNK)])


@jax.jit
def _svd_predict(user2d, item2d, tbl, Sigma, user_bias, item_bias):
  mesh = plsc.VectorSubcoreMesh(core_axis_name="c", subcore_axis_name="s",
                                num_cores=NC, num_subcores=NS)
  return pl.kernel(
      _body,
      out_type=jax.ShapeDtypeStruct((B,), jnp.float32),
      mesh=mesh,
      compiler_params=pltpu.CompilerParams(needs_layout_passes=False,
                                           use_tc_tiling_on_sc=False),
      scratch_types=[
          pltpu.VMEM((QROWS, 128), jnp.int32),
          pltpu.VMEM((QROWS, 128), jnp.int32),
          pltpu.VMEM((QROWS, 128), jnp.int32),
          pltpu.VMEM((CHUNK, D), jnp.float32),
          pltpu.VMEM((CHUNK, D), jnp.float32),
          pltpu.VMEM((CHUNK * D,), jnp.float32),
          pltpu.VMEM((CHUNK * D,), jnp.float32),
          pltpu.VMEM((QROWS, 128), jnp.float32),
          pltpu.VMEM((QROWS, 128), jnp.float32),
          pltpu.VMEM((D,), jnp.float32),
          pltpu.VMEM((CHUNK,), jnp.float32),
          pltpu.SemaphoreType.DMA,
      ],
  )(user2d, item2d, tbl, Sigma, user_bias, item_bias)


def kernel(user, item, U, Sigma, VT, user_bias, item_bias):
  user2d = user.reshape(B // 128, 128)
  item2d = item.reshape(B // 128, 128)
  tbl = jnp.concatenate([U, VT.T], axis=0)
  return _svd_predict(user2d, item2d, tbl, Sigma, user_bias, item_bias)


# speedup vs baseline: 1.2792x; 1.2792x over previous
import jax
import jax.numpy as jnp
from jax import lax
from jax.experimental import pallas as pl
from jax.experimental.pallas import tpu as pltpu
from jax.experimental.pallas import tpu_sc as plsc

B = 16384
D = 32
NC = 2
NS = 16
NW = NC * NS
CHUNK = B // NW
QROWS = CHUNK // 128
NITEMS = 1_000_000


def _body(user_hbm, item_hbm, u_hbm, sig_hbm, v_hbm, ub_hbm, ib_hbm,
          out_hbm, uidx, iidx, urows, vrows, ud, vtd, ubv, ibv, sig, outv,
          sem):
  wid = lax.axis_index("s") * NC + lax.axis_index("c")
  r0 = wid * QROWS
  base = wid * CHUNK
  iota = lax.iota(jnp.int32, 16)

  pltpu.sync_copy(user_hbm.at[pl.ds(r0, QROWS)], uidx)
  pltpu.sync_copy(item_hbm.at[pl.ds(r0, QROWS)], iidx)
  pltpu.sync_copy(sig_hbm, sig)

  copies = []
  for q in range(QROWS):
    copies.append(pltpu.async_copy(ub_hbm.at[uidx.at[q]], ubv.at[q], sem))
    copies.append(pltpu.async_copy(ib_hbm.at[iidx.at[q]], ibv.at[q], sem))
    copies.append(
        pltpu.async_copy(u_hbm.at[uidx.at[q]],
                         urows.at[pl.ds(q * 128, 128)], sem))
    copies.append(
        pltpu.async_copy(v_hbm.at[iidx.at[q]],
                         vrows.at[pl.ds(q * 128, 128)], sem))
  for cp in copies:
    cp.wait()

  dvec = iota * CHUNK

  def transpose(j, _):
    for h in range(2):
      idx = dvec + (h * 16 * CHUNK + j)
      plsc.store_scatter(ud, [idx], urows[j, pl.ds(h * 16, 16)])
      plsc.store_scatter(vtd, [idx], vrows[j, pl.ds(h * 16, 16)])
    return 0

  lax.fori_loop(0, CHUNK, transpose, 0)

  def compute(jc, _):
    row = jc // 8
    col = (jc % 8) * 16
    sig_lo = sig[pl.ds(0, 16)]
    sig_hi = sig[pl.ds(16, 16)]
    acc = ubv[row, pl.ds(col, 16)] + ibv[row, pl.ds(col, 16)]
    for d in range(D):
      sig_d = sig_lo[d] if d < 16 else sig_hi[d - 16]
      vt_chunk = vtd[pl.ds(d * CHUNK + jc * 16, 16)]
      u_chunk = ud[pl.ds(d * CHUNK + jc * 16, 16)]
      acc = acc + (sig_d * vt_chunk) * u_chunk
    outv[pl.ds(jc * 16, 16)] = acc
    return 0

  lax.fori_loop(0, CHUNK // 16, compute, 0)

  pltpu.sync_copy(outv, out_hbm.at[pl.ds(base, CHUNK)])


@jax.jit
def _svd_predict(user2d, item2d, U, Sigma, V, user_bias, item_bias):
  mesh = plsc.VectorSubcoreMesh(core_axis_name="c", subcore_axis_name="s",
                                num_cores=NC, num_subcores=NS)
  return pl.kernel(
      _body,
      out_type=jax.ShapeDtypeStruct((B,), jnp.float32),
      mesh=mesh,
      compiler_params=pltpu.CompilerParams(needs_layout_passes=False,
                                           use_tc_tiling_on_sc=False),
      scratch_types=[
          pltpu.VMEM((QROWS, 128), jnp.int32),
          pltpu.VMEM((QROWS, 128), jnp.int32),
          pltpu.VMEM((CHUNK, D), jnp.float32),
          pltpu.VMEM((CHUNK, D), jnp.float32),
          pltpu.VMEM((CHUNK * D,), jnp.float32),
          pltpu.VMEM((CHUNK * D,), jnp.float32),
          pltpu.VMEM((QROWS, 128), jnp.float32),
          pltpu.VMEM((QROWS, 128), jnp.float32),
          pltpu.VMEM((D,), jnp.float32),
          pltpu.VMEM((CHUNK,), jnp.float32),
          pltpu.SemaphoreType.DMA,
      ],
  )(user2d, item2d, U, Sigma, V, user_bias, item_bias)


def kernel(user, item, U, Sigma, VT, user_bias, item_bias):
  user2d = user.reshape(B // 128, 128)
  item2d = item.reshape(B // 128, 128)
  return _svd_predict(user2d, item2d, U, Sigma, VT.T, user_bias, item_bias)
